# baseline (speedup 1.0000x reference)
import functools

import jax
import jax.numpy as jnp
from jax import lax
from jax.experimental import pallas as pl
from jax.experimental.pallas import tpu as pltpu
from jax.experimental.pallas import tpu_sc as plsc

N_NODES = 10000
D = 256
DH = 128
N_PAD = N_NODES + 8
CNT_W = 128
EDGE_CHUNK = 128
NT = 16


def _prep_edges(edge_index):
    src = edge_index[0]
    dst = edge_index[1]
    e = src.shape[0]
    per = 2 * NT * EDGE_CHUNK
    e_pad = ((e + per - 1) // per) * per
    pad = e_pad - e
    if pad:
        src = jnp.concatenate([src, jnp.zeros((pad,), jnp.int32)])
        dst = jnp.concatenate([dst, jnp.full((pad,), N_NODES, jnp.int32)])
    ch = e_pad // (NT * EDGE_CHUNK)
    return src.reshape(NT, ch, EDGE_CHUNK), dst.reshape(NT, ch, EDGE_CHUNK), ch


@functools.cache
def _make_aggregate(ch):
    mesh = plsc.VectorSubcoreMesh(core_axis_name="c", subcore_axis_name="s")
    out_type = [
        jax.ShapeDtypeStruct((2, N_PAD, DH), jnp.float32),
    ]
    scratch = [
        pltpu.VMEM((ch, EDGE_CHUNK), jnp.int32),
        pltpu.VMEM((ch, EDGE_CHUNK), jnp.int32),
        pltpu.VMEM((EDGE_CHUNK, DH), jnp.float32),
        pltpu.VMEM_SHARED((N_PAD, DH), jnp.float32),
    ]

    @functools.partial(pl.kernel, out_type=out_type, mesh=mesh,
                       scratch_types=scratch)
    def agg(x_st, srcr, dstr, zeros_feat,
            out, src_v, dst_v, rows, acc):
        c = lax.axis_index("c")
        s = lax.axis_index("s")

        @pl.when(s == 0)
        def _():
            pltpu.sync_copy(zeros_feat, acc)

        pltpu.sync_copy(srcr.at[s], src_v)
        pltpu.sync_copy(dstr.at[s], dst_v)
        plsc.subcore_barrier()

        @pl.loop(0, ch)
        def _(j):
            pltpu.sync_copy(x_st.at[c].at[src_v.at[j]], rows)
            pltpu.sync_copy(rows, acc.at[dst_v.at[j]], add=True)

        plsc.subcore_barrier()

        @pl.when(s == 0)
        def _():
            pltpu.sync_copy(acc, out.at[c])

    return agg


@functools.cache
def _make_counts(ch):
    mesh = plsc.VectorSubcoreMesh(core_axis_name="c", subcore_axis_name="s")
    out_type = [
        jax.ShapeDtypeStruct((N_PAD, CNT_W), jnp.float32),
        jax.ShapeDtypeStruct((N_PAD, CNT_W), jnp.float32),
    ]
    scratch = [
        pltpu.VMEM((ch, EDGE_CHUNK), jnp.int32),
        pltpu.VMEM((EDGE_CHUNK, CNT_W), jnp.float32),
        pltpu.VMEM_SHARED((N_PAD, CNT_W), jnp.float32),
    ]

    @functools.partial(pl.kernel, out_type=out_type, mesh=mesh,
                       scratch_types=scratch)
    def cnt(dst1r, dst2r, zeros_cnt, ones_hbm,
            out_c1, out_c2, dst_v, ones_v, cacc):
        c = lax.axis_index("c")
        s = lax.axis_index("s")

        @pl.when(s == 0)
        def _():
            pltpu.sync_copy(zeros_cnt, cacc)

        @pl.when(c == 0)
        def _():
            pltpu.sync_copy(dst1r.at[s], dst_v)

        @pl.when(c == 1)
        def _():
            pltpu.sync_copy(dst2r.at[s], dst_v)

        pltpu.sync_copy(ones_hbm, ones_v)
        plsc.subcore_barrier()

        @pl.loop(0, ch)
        def _(j):
            pltpu.sync_copy(ones_v, cacc.at[dst_v.at[j]], add=True)

        plsc.subcore_barrier()

        @pl.when((s == 0) & (c == 0))
        def _():
            pltpu.sync_copy(cacc, out_c1)

        @pl.when((s == 0) & (c == 1))
        def _():
            pltpu.sync_copy(cacc, out_c2)

    return cnt


_R = 1000


def _layer1_body(slo_ref, shi_ref, cnt_ref, x_ref, wlt_ref, wlb_ref, wr_ref,
                 b_ref, hst_ref):
    inv = 1.0 / jnp.maximum(cnt_ref[:, 0:1], 1.0)
    h = (jnp.dot(slo_ref[0] * inv, wlt_ref[...],
                 preferred_element_type=jnp.float32)
         + jnp.dot(shi_ref[0] * inv, wlb_ref[...],
                   preferred_element_type=jnp.float32)
         + jnp.dot(x_ref[...], wr_ref[...], preferred_element_type=jnp.float32)
         + b_ref[...])
    h = jnp.maximum(h, 0.0)
    hst_ref[0] = h[:, :DH]
    hst_ref[1] = h[:, DH:]


def _tc_layer1(s_st, cnt, x, wl_t, wl_b, wr, b):
    return pl.pallas_call(
        _layer1_body,
        grid=(N_NODES // _R,),
        in_specs=[
            pl.BlockSpec((1, _R, DH), lambda i: (0, i, 0)),
            pl.BlockSpec((1, _R, DH), lambda i: (1, i, 0)),
            pl.BlockSpec((_R, CNT_W), lambda i: (i, 0)),
            pl.BlockSpec((_R, D), lambda i: (i, 0)),
            pl.BlockSpec((DH, D), lambda i: (0, 0)),
            pl.BlockSpec((DH, D), lambda i: (0, 0)),
            pl.BlockSpec((D, D), lambda i: (0, 0)),
            pl.BlockSpec((1, D), lambda i: (0, 0)),
        ],
        out_specs=[pl.BlockSpec((2, _R, DH), lambda i: (0, i, 0))],
        out_shape=[jax.ShapeDtypeStruct((2, N_NODES, DH), jnp.float32)],
    )(s_st, s_st, cnt, x, wl_t, wl_b, wr, b)[0]


def _head_body(slo_ref, shi_ref, cnt_ref, hlo_ref, hhi_ref, flat_ref,
               wl2_ref, wr2_ref, bl2_ref, wf_ref, bf_ref, woh_ref, wof_ref,
               bo_ref, out_ref):
    f32 = jnp.float32
    woh = woh_ref[...]
    wlh = jnp.dot(wl2_ref[...], woh, preferred_element_type=f32)
    wrh = jnp.dot(wr2_ref[...], woh, preferred_element_type=f32)
    wff = jnp.dot(wf_ref[...], wof_ref[...], preferred_element_type=f32)
    bias = (jnp.dot(bl2_ref[...], woh, preferred_element_type=f32)
            + jnp.dot(bf_ref[...], wof_ref[...], preferred_element_type=f32)
            + bo_ref[...])
    inv = 1.0 / jnp.maximum(cnt_ref[:, 0:1], 1.0)
    out_ref[...] = (
        jnp.dot(slo_ref[0] * inv, wlh[:DH], preferred_element_type=f32)
        + jnp.dot(shi_ref[0] * inv, wlh[DH:], preferred_element_type=f32)
        + jnp.dot(hlo_ref[0], wrh[:DH], preferred_element_type=f32)
        + jnp.dot(hhi_ref[0], wrh[DH:], preferred_element_type=f32)
        + jnp.dot(flat_ref[...], wff, preferred_element_type=f32)
        + bias)


def _tc_head(s2_st, cnt, h_st, flat, wl2, wr2, bl2, wf, bf,
             wo_h, wo_f, bo, n_cls, d_flat, flat_hid):
    return pl.pallas_call(
        _head_body,
        grid=(N_NODES // _R,),
        in_specs=[
            pl.BlockSpec((1, _R, DH), lambda i: (0, i, 0)),
            pl.BlockSpec((1, _R, DH), lambda i: (1, i, 0)),
            pl.BlockSpec((_R, CNT_W), lambda i: (i, 0)),
            pl.BlockSpec((1, _R, DH), lambda i: (0, i, 0)),
            pl.BlockSpec((1, _R, DH), lambda i: (1, i, 0)),
            pl.BlockSpec((_R, d_flat), lambda i: (i, 0)),
            pl.BlockSpec((D, D), lambda i: (0, 0)),
            pl.BlockSpec((D, D), lambda i: (0, 0)),
            pl.BlockSpec((1, D), lambda i: (0, 0)),
            pl.BlockSpec((d_flat, flat_hid), lambda i: (0, 0)),
            pl.BlockSpec((1, flat_hid), lambda i: (0, 0)),
            pl.BlockSpec((D, n_cls), lambda i: (0, 0)),
            pl.BlockSpec((flat_hid, n_cls), lambda i: (0, 0)),
            pl.BlockSpec((1, n_cls), lambda i: (0, 0)),
        ],
        out_specs=[pl.BlockSpec((_R, n_cls), lambda i: (i, 0))],
        out_shape=[jax.ShapeDtypeStruct((N_NODES, n_cls), jnp.float32)],
    )(s2_st, s2_st, cnt, h_st, h_st, flat, wl2, wr2, bl2, wf, bf, wo_h, wo_f,
      bo)[0]


def kernel(x, flat, edge_index_layer1, edge_index_layer2,
           W_l1, b_l1, W_r1, W_l2, b_l2, W_r2,
           W_flat, b_flat, W_out, b_out):
    src1, dst1, ch1 = _prep_edges(edge_index_layer1)
    src2, dst2, ch2 = _prep_edges(edge_index_layer2)
    zeros_feat = jnp.zeros((N_PAD, DH), jnp.float32)
    zeros_cnt = jnp.zeros((N_PAD, CNT_W), jnp.float32)
    ones_hbm = jnp.ones((EDGE_CHUNK, CNT_W), jnp.float32)

    assert ch1 == ch2
    cnt1, cnt2 = _make_counts(ch1)(dst1, dst2, zeros_cnt, ones_hbm)

    x_st = jnp.stack([x[:, :DH], x[:, DH:]])
    agg = _make_aggregate(ch1)
    s1 = agg(x_st, src1, dst1, zeros_feat)[0]

    h_st = _tc_layer1(s1, cnt1, x, W_l1[:DH], W_l1[DH:], W_r1, b_l1[None])

    s2 = agg(h_st, src2, dst2, zeros_feat)[0]

    n_cls = b_out.shape[0]
    d_flat = flat.shape[1]
    flat_hid = b_flat.shape[0]
    return _tc_head(s2, cnt2, h_st, flat,
                    W_l2, W_r2, b_l2[None], W_flat, b_flat[None],
                    W_out[:D], W_out[D:], b_out[None],
                    n_cls, d_flat, flat_hid)


# speedup vs baseline: 1.3035x; 1.3035x over previous
import functools

import jax
import jax.numpy as jnp
from jax import lax
from jax.experimental import pallas as pl
from jax.experimental.pallas import tpu as pltpu
from jax.experimental.pallas import tpu_sc as plsc

N_NODES = 10000
D = 256
DH = 128
N_PAD = N_NODES + 8
CNT_W = 128
EDGE_CHUNK = 128
NT = 16


def _prep_edges(edge_index):
    src = edge_index[0]
    dst = edge_index[1]
    e = src.shape[0]
    per = NT * EDGE_CHUNK
    e_pad = ((e + per - 1) // per) * per
    pad = e_pad - e
    if pad:
        src = jnp.concatenate([src, jnp.zeros((pad,), jnp.int32)])
        dst = jnp.concatenate([dst, jnp.full((pad,), N_NODES, jnp.int32)])
    ch = e_pad // per
    return src.reshape(NT, ch, EDGE_CHUNK), dst.reshape(NT, ch, EDGE_CHUNK), ch


@functools.cache
def _make_aggregate(ch):
    mesh = plsc.VectorSubcoreMesh(core_axis_name="c", subcore_axis_name="s")
    out_type = [
        jax.ShapeDtypeStruct((N_PAD, DH), jnp.float32),
        jax.ShapeDtypeStruct((N_PAD, DH), jnp.float32),
    ]
    scratch = [
        pltpu.VMEM((ch, EDGE_CHUNK), jnp.int32),
        pltpu.VMEM((ch, EDGE_CHUNK), jnp.int32),
        pltpu.VMEM((EDGE_CHUNK, DH), jnp.float32),
        pltpu.VMEM_SHARED((N_PAD, DH), jnp.float32),
    ]

    @functools.partial(pl.kernel, out_type=out_type, mesh=mesh,
                       scratch_types=scratch)
    def agg(x_lo, x_hi, srcr, dstr, zeros_feat,
            out_lo, out_hi, src_v, dst_v, rows, acc):
        c = lax.axis_index("c")
        s = lax.axis_index("s")

        @pl.when(s == 0)
        def _():
            pltpu.sync_copy(zeros_feat, acc)

        pltpu.sync_copy(srcr.at[s], src_v)
        pltpu.sync_copy(dstr.at[s], dst_v)
        plsc.subcore_barrier()

        @pl.loop(0, ch)
        def _(j):
            @pl.when(c == 0)
            def _():
                pltpu.sync_copy(x_lo.at[src_v.at[j]], rows)

            @pl.when(c == 1)
            def _():
                pltpu.sync_copy(x_hi.at[src_v.at[j]], rows)

            pltpu.sync_copy(rows, acc.at[dst_v.at[j]], add=True)

        plsc.subcore_barrier()

        @pl.when((s == 0) & (c == 0))
        def _():
            pltpu.sync_copy(acc, out_lo)

        @pl.when((s == 0) & (c == 1))
        def _():
            pltpu.sync_copy(acc, out_hi)

    return agg


@functools.cache
def _make_counts(ch):
    mesh = plsc.VectorSubcoreMesh(core_axis_name="c", subcore_axis_name="s")
    out_type = [
        jax.ShapeDtypeStruct((N_PAD, CNT_W), jnp.float32),
        jax.ShapeDtypeStruct((N_PAD, CNT_W), jnp.float32),
    ]
    scratch = [
        pltpu.VMEM((ch, EDGE_CHUNK), jnp.int32),
        pltpu.VMEM((EDGE_CHUNK, CNT_W), jnp.float32),
        pltpu.VMEM_SHARED((N_PAD, CNT_W), jnp.float32),
    ]

    @functools.partial(pl.kernel, out_type=out_type, mesh=mesh,
                       scratch_types=scratch)
    def cnt(dst1r, dst2r, zeros_cnt, ones_hbm,
            out_c1, out_c2, dst_v, ones_v, cacc):
        c = lax.axis_index("c")
        s = lax.axis_index("s")

        @pl.when(s == 0)
        def _():
            pltpu.sync_copy(zeros_cnt, cacc)

        @pl.when(c == 0)
        def _():
            pltpu.sync_copy(dst1r.at[s], dst_v)

        @pl.when(c == 1)
        def _():
            pltpu.sync_copy(dst2r.at[s], dst_v)

        pltpu.sync_copy(ones_hbm, ones_v)
        plsc.subcore_barrier()

        @pl.loop(0, ch)
        def _(j):
            pltpu.sync_copy(ones_v, cacc.at[dst_v.at[j]], add=True)

        plsc.subcore_barrier()

        @pl.when((s == 0) & (c == 0))
        def _():
            pltpu.sync_copy(cacc, out_c1)

        @pl.when((s == 0) & (c == 1))
        def _():
            pltpu.sync_copy(cacc, out_c2)

    return cnt


_R = 1000


def _layer1_body(slo_ref, shi_ref, cnt_ref, x_ref, wlt_ref, wlb_ref, wr_ref,
                 b_ref, hlo_ref, hhi_ref):
    inv = 1.0 / jnp.maximum(cnt_ref[:, 0:1], 1.0)
    h = (jnp.dot(slo_ref[...] * inv, wlt_ref[...],
                 preferred_element_type=jnp.float32)
         + jnp.dot(shi_ref[...] * inv, wlb_ref[...],
                   preferred_element_type=jnp.float32)
         + jnp.dot(x_ref[...], wr_ref[...], preferred_element_type=jnp.float32)
         + b_ref[...])
    h = jnp.maximum(h, 0.0)
    hlo_ref[...] = h[:, :DH]
    hhi_ref[...] = h[:, DH:]


def _tc_layer1(s_lo, s_hi, cnt, x, wl_t, wl_b, wr, b):
    return pl.pallas_call(
        _layer1_body,
        grid=(N_NODES // _R,),
        in_specs=[
            pl.BlockSpec((_R, DH), lambda i: (i, 0)),
            pl.BlockSpec((_R, DH), lambda i: (i, 0)),
            pl.BlockSpec((_R, CNT_W), lambda i: (i, 0)),
            pl.BlockSpec((_R, D), lambda i: (i, 0)),
            pl.BlockSpec((DH, D), lambda i: (0, 0)),
            pl.BlockSpec((DH, D), lambda i: (0, 0)),
            pl.BlockSpec((D, D), lambda i: (0, 0)),
            pl.BlockSpec((1, D), lambda i: (0, 0)),
        ],
        out_specs=[pl.BlockSpec((_R, DH), lambda i: (i, 0)),
                   pl.BlockSpec((_R, DH), lambda i: (i, 0))],
        out_shape=[jax.ShapeDtypeStruct((N_NODES, DH), jnp.float32),
                   jax.ShapeDtypeStruct((N_NODES, DH), jnp.float32)],
    )(s_lo, s_hi, cnt, x, wl_t, wl_b, wr, b)


def _head_body(slo_ref, shi_ref, cnt_ref, hlo_ref, hhi_ref, flat_ref,
               wl2_ref, wr2_ref, bl2_ref, wf_ref, bf_ref, woh_ref, wof_ref,
               bo_ref, out_ref):
    f32 = jnp.float32
    woh = woh_ref[...]
    wlh = jnp.dot(wl2_ref[...], woh, preferred_element_type=f32)
    wrh = jnp.dot(wr2_ref[...], woh, preferred_element_type=f32)
    wff = jnp.dot(wf_ref[...], wof_ref[...], preferred_element_type=f32)
    bias = (jnp.dot(bl2_ref[...], woh, preferred_element_type=f32)
            + jnp.dot(bf_ref[...], wof_ref[...], preferred_element_type=f32)
            + bo_ref[...])
    inv = 1.0 / jnp.maximum(cnt_ref[:, 0:1], 1.0)
    out_ref[...] = (
        jnp.dot(slo_ref[...] * inv, wlh[:DH], preferred_element_type=f32)
        + jnp.dot(shi_ref[...] * inv, wlh[DH:], preferred_element_type=f32)
        + jnp.dot(hlo_ref[...], wrh[:DH], preferred_element_type=f32)
        + jnp.dot(hhi_ref[...], wrh[DH:], preferred_element_type=f32)
        + jnp.dot(flat_ref[...], wff, preferred_element_type=f32)
        + bias)


def _tc_head(s_lo, s_hi, cnt, h_lo, h_hi, flat, wl2, wr2, bl2, wf, bf,
             wo_h, wo_f, bo, n_cls, d_flat, flat_hid):
    return pl.pallas_call(
        _head_body,
        grid=(N_NODES // _R,),
        in_specs=[
            pl.BlockSpec((_R, DH), lambda i: (i, 0)),
            pl.BlockSpec((_R, DH), lambda i: (i, 0)),
            pl.BlockSpec((_R, CNT_W), lambda i: (i, 0)),
            pl.BlockSpec((_R, DH), lambda i: (i, 0)),
            pl.BlockSpec((_R, DH), lambda i: (i, 0)),
            pl.BlockSpec((_R, d_flat), lambda i: (i, 0)),
            pl.BlockSpec((D, D), lambda i: (0, 0)),
            pl.BlockSpec((D, D), lambda i: (0, 0)),
            pl.BlockSpec((1, D), lambda i: (0, 0)),
            pl.BlockSpec((d_flat, flat_hid), lambda i: (0, 0)),
            pl.BlockSpec((1, flat_hid), lambda i: (0, 0)),
            pl.BlockSpec((D, n_cls), lambda i: (0, 0)),
            pl.BlockSpec((flat_hid, n_cls), lambda i: (0, 0)),
            pl.BlockSpec((1, n_cls), lambda i: (0, 0)),
        ],
        out_specs=[pl.BlockSpec((_R, n_cls), lambda i: (i, 0))],
        out_shape=[jax.ShapeDtypeStruct((N_NODES, n_cls), jnp.float32)],
    )(s_lo, s_hi, cnt, h_lo, h_hi, flat, wl2, wr2, bl2, wf, bf, wo_h, wo_f,
      bo)[0]


def kernel(x, flat, edge_index_layer1, edge_index_layer2,
           W_l1, b_l1, W_r1, W_l2, b_l2, W_r2,
           W_flat, b_flat, W_out, b_out):
    src1, dst1, ch1 = _prep_edges(edge_index_layer1)
    src2, dst2, ch2 = _prep_edges(edge_index_layer2)
    zeros_feat = jnp.zeros((N_PAD, DH), jnp.float32)
    zeros_cnt = jnp.zeros((N_PAD, CNT_W), jnp.float32)
    ones_hbm = jnp.ones((EDGE_CHUNK, CNT_W), jnp.float32)

    assert ch1 == ch2
    cnt1, cnt2 = _make_counts(ch1)(dst1, dst2, zeros_cnt, ones_hbm)

    x_lo = x[:, :DH]
    x_hi = x[:, DH:]
    agg1 = _make_aggregate(ch1)
    s_lo, s_hi = agg1(x_lo, x_hi, src1, dst1, zeros_feat)

    h_lo, h_hi = _tc_layer1(s_lo, s_hi, cnt1, x, W_l1[:DH], W_l1[DH:], W_r1,
                            b_l1[None])

    agg2 = _make_aggregate(ch2)
    s2_lo, s2_hi = agg2(h_lo, h_hi, src2, dst2, zeros_feat)

    n_cls = b_out.shape[0]
    d_flat = flat.shape[1]
    flat_hid = b_flat.shape[0]
    return _tc_head(s2_lo, s2_hi, cnt2, h_lo, h_hi, flat,
                    W_l2, W_r2, b_l2[None], W_flat, b_flat[None],
                    W_out[:D], W_out[D:], b_out[None],
                    n_cls, d_flat, flat_hid)
